# baseline (speedup 1.0000x reference)
import functools

import jax
import jax.numpy as jnp
from jax import lax
from jax.experimental import pallas as pl
from jax.experimental.pallas import tpu as pltpu
from jax.experimental.pallas import tpu_sc as plsc

D_U = 1000000
D_V = 100000
D_K = 64
NNZ = 26
NC = 2
NS = 16
NW = NC * NS
ROWS_PER_CHUNK = 4
IDX_PER_CHUNK = ROWS_PER_CHUNK * NNZ
NBUF = 8

B_U = 16384
B_V = 4096
BW_U = B_U // NW
BW_V = B_V // NW
CH_U = BW_U // ROWS_PER_CHUNK
CH_V = BW_V // ROWS_PER_CHUNK

FLAT_BLK = 8192


def _flatten_body(x_ref, o_ref):
    xt = x_ref[...].T
    h = FLAT_BLK // 2
    o_ref[...] = jnp.concatenate([xt[:h, :], xt[h:, :]], axis=1)


def _make_flatten(cols):
    grid = (cols + FLAT_BLK - 1) // FLAT_BLK
    return pl.pallas_call(
        _flatten_body,
        grid=(grid,),
        in_specs=[pl.BlockSpec((D_K, FLAT_BLK), lambda g: (0, g))],
        out_specs=pl.BlockSpec((FLAT_BLK // 2, 2 * D_K), lambda g: (g, 0)),
        out_shape=jax.ShapeDtypeStruct((grid * FLAT_BLK // 2, 2 * D_K), jnp.float32),
    )


def _phi(idx):
    h = FLAT_BLK // 2
    a = idx & (FLAT_BLK - 1)
    g = idx >> 13
    return (g << 13) + jnp.where(a < h, 2 * a, 2 * (a - h) + 1)


def _padded_rows(cols):
    grid = (cols + FLAT_BLK - 1) // FLAT_BLK
    return grid * FLAT_BLK


_FLAT_U = _make_flatten(D_U)
_FLAT_V = _make_flatten(D_V)


def _make_kernel(batch):
    bw = batch // NW
    n_chunks = bw // ROWS_PER_CHUNK
    nidx = bw * NNZ
    mesh = plsc.VectorSubcoreMesh(core_axis_name="c", subcore_axis_name="s")

    @functools.partial(
        pl.kernel,
        out_type=jax.ShapeDtypeStruct((batch, D_K), jnp.float32),
        mesh=mesh,
        compiler_params=pltpu.CompilerParams(use_tc_tiling_on_sc=False),
        scratch_types=[
            pltpu.VMEM((nidx,), jnp.int32),
            pltpu.VMEM((NBUF, IDX_PER_CHUNK, D_K), jnp.float32),
            pltpu.VMEM((bw, D_K), jnp.float32),
            pltpu.SemaphoreType.DMA((NBUF,)),
        ],
    )
    def fm2(idx_hbm, tbl_hbm, out_hbm, idx1d_v, bufs_v, out_v, sems):
        wid = lax.axis_index("s") * NC + lax.axis_index("c")
        pltpu.sync_copy(
            idx_hbm.at[pl.ds(wid * nidx, nidx)], idx1d_v
        )

        def start(g, b):
            pltpu.async_copy(
                tbl_hbm.at[idx1d_v.at[pl.ds(g * IDX_PER_CHUNK, IDX_PER_CHUNK)]],
                bufs_v.at[b],
                sems.at[b],
            )

        for b in range(NBUF - 1):
            start(b, b)

        def outer_body(go, carry):
            for b in range(NBUF):
                g = go * NBUF + b
                s = g + NBUF - 1
                sb = (b + NBUF - 1) % NBUF

                @pl.when(s < n_chunks)
                def _():
                    start(s, sb)

                pltpu.make_async_copy(
                    tbl_hbm.at[idx1d_v.at[pl.ds(g * IDX_PER_CHUNK, IDX_PER_CHUNK)]],
                    bufs_v.at[b],
                    sems.at[b],
                ).wait()
                for r in range(ROWS_PER_CHUNK):
                    row = g * ROWS_PER_CHUNK + r
                    for v in range(D_K // 16):
                        sl = pl.ds(v * 16, 16)
                        accs = [bufs_v[b, r * NNZ + j, sl] for j in range(4)]
                        for j in range(4, NNZ):
                            accs[j % 4] = accs[j % 4] + bufs_v[b, r * NNZ + j, sl]
                        out_v[row, sl] = (accs[0] + accs[1]) + (accs[2] + accs[3])
            return carry

        lax.fori_loop(0, n_chunks // NBUF, outer_body, 0)
        pltpu.sync_copy(out_v, out_hbm.at[pl.ds(wid * bw, bw)])

    return fm2


_FM2_U = _make_kernel(B_U)
_FM2_V = _make_kernel(B_V)


@jax.jit
def kernel(U, V, W_u, W_v):
    u_flat = lax.optimization_barrier(_phi(U.astype(jnp.int32)).reshape(-1))
    v_flat = lax.optimization_barrier(_phi(V.astype(jnp.int32)).reshape(-1))
    wu_lin = _FLAT_U(W_u.T).reshape(_padded_rows(D_U), D_K)
    wv_lin = _FLAT_V(W_v.T).reshape(_padded_rows(D_V), D_K)
    return _FM2_U(u_flat, wu_lin), _FM2_V(v_flat, wv_lin)


# speedup vs baseline: 1.0619x; 1.0619x over previous
import functools

import jax
import jax.numpy as jnp
from jax import lax
from jax.experimental import pallas as pl
from jax.experimental.pallas import tpu as pltpu
from jax.experimental.pallas import tpu_sc as plsc

D_U = 1000000
D_V = 100000
D_K = 64
NNZ = 26
NC = 2
NS = 16
NW = NC * NS
ROWS_PER_CHUNK = 4
IDX_PER_CHUNK = ROWS_PER_CHUNK * NNZ
NBUF = 4

B_U = 16384
B_V = 4096
BW_U = B_U // NW
BW_V = B_V // NW
CH_U = BW_U // ROWS_PER_CHUNK
CH_V = BW_V // ROWS_PER_CHUNK

FLAT_BLK = 8192


def _flatten_body(x_ref, o_ref):
    xt = x_ref[...].T
    h = FLAT_BLK // 2
    o_ref[...] = jnp.concatenate([xt[:h, :], xt[h:, :]], axis=1)


def _make_flatten(cols):
    grid = (cols + FLAT_BLK - 1) // FLAT_BLK
    return pl.pallas_call(
        _flatten_body,
        grid=(grid,),
        in_specs=[pl.BlockSpec((D_K, FLAT_BLK), lambda g: (0, g))],
        out_specs=pl.BlockSpec((FLAT_BLK // 2, 2 * D_K), lambda g: (g, 0)),
        out_shape=jax.ShapeDtypeStruct((grid * FLAT_BLK // 2, 2 * D_K), jnp.float32),
    )


def _phi(idx):
    h = FLAT_BLK // 2
    a = idx & (FLAT_BLK - 1)
    g = idx >> 13
    return (g << 13) + jnp.where(a < h, 2 * a, 2 * (a - h) + 1)


def _padded_rows(cols):
    grid = (cols + FLAT_BLK - 1) // FLAT_BLK
    return grid * FLAT_BLK


_FLAT_U = _make_flatten(D_U)
_FLAT_V = _make_flatten(D_V)


def _make_kernel(batch):
    bw = batch // NW
    n_chunks = bw // ROWS_PER_CHUNK
    nidx = bw * NNZ
    mesh = plsc.VectorSubcoreMesh(core_axis_name="c", subcore_axis_name="s")

    @functools.partial(
        pl.kernel,
        out_type=jax.ShapeDtypeStruct((batch, D_K), jnp.float32),
        mesh=mesh,
        compiler_params=pltpu.CompilerParams(use_tc_tiling_on_sc=False),
        scratch_types=[
            pltpu.VMEM((nidx,), jnp.int32),
            pltpu.VMEM((NBUF, IDX_PER_CHUNK, D_K), jnp.float32),
            pltpu.VMEM((bw, D_K), jnp.float32),
            pltpu.SemaphoreType.DMA((NBUF,)),
        ],
    )
    def fm2(idx_hbm, tbl_hbm, out_hbm, idx1d_v, bufs_v, out_v, sems):
        wid = lax.axis_index("s") * NC + lax.axis_index("c")
        pltpu.sync_copy(
            idx_hbm.at[pl.ds(wid * nidx, nidx)], idx1d_v
        )

        def start(g, b):
            pltpu.async_copy(
                tbl_hbm.at[idx1d_v.at[pl.ds(g * IDX_PER_CHUNK, IDX_PER_CHUNK)]],
                bufs_v.at[b],
                sems.at[b],
            )

        for b in range(NBUF - 1):
            start(b, b)

        def outer_body(go, carry):
            for b in range(NBUF):
                g = go * NBUF + b
                s = g + NBUF - 1
                sb = (b + NBUF - 1) % NBUF

                @pl.when(s < n_chunks)
                def _():
                    start(s, sb)

                pltpu.make_async_copy(
                    tbl_hbm.at[idx1d_v.at[pl.ds(g * IDX_PER_CHUNK, IDX_PER_CHUNK)]],
                    bufs_v.at[b],
                    sems.at[b],
                ).wait()
                for r in range(ROWS_PER_CHUNK):
                    row = g * ROWS_PER_CHUNK + r
                    for v in range(D_K // 16):
                        sl = pl.ds(v * 16, 16)
                        accs = [bufs_v[b, r * NNZ + j, sl] for j in range(4)]
                        for j in range(4, NNZ):
                            accs[j % 4] = accs[j % 4] + bufs_v[b, r * NNZ + j, sl]
                        out_v[row, sl] = (accs[0] + accs[1]) + (accs[2] + accs[3])
            return carry

        lax.fori_loop(0, n_chunks // NBUF, outer_body, 0)
        pltpu.sync_copy(out_v, out_hbm.at[pl.ds(wid * bw, bw)])

    return fm2


_FM2_U = _make_kernel(B_U)
_FM2_V = _make_kernel(B_V)


@jax.jit
def kernel(U, V, W_u, W_v):
    u_flat = lax.optimization_barrier(_phi(U.astype(jnp.int32)).reshape(-1))
    v_flat = lax.optimization_barrier(_phi(V.astype(jnp.int32)).reshape(-1))
    wu_lin = _FLAT_U(W_u.T).reshape(_padded_rows(D_U), D_K)
    wv_lin = _FLAT_V(W_v.T).reshape(_padded_rows(D_V), D_K)
    return _FM2_U(u_flat, wu_lin), _FM2_V(v_flat, wv_lin)
